# baseline (speedup 1.0000x reference)
import functools
import math

import jax
import jax.numpy as jnp
from jax import lax
from jax.experimental import pallas as pl
from jax.experimental.pallas import tpu as pltpu
from jax.experimental.pallas import tpu_sc as plsc

D = 32
NC, NS, L = 2, 16, 16
NW = NC * NS
C = 128

_SCALE = math.sqrt(float(D))


@functools.cache
def _make_gather(B):
    assert B % (NW * C) == 0
    b_per_w = B // NW
    n_chunks = b_per_w // C
    mesh = plsc.VectorSubcoreMesh(core_axis_name="c", subcore_axis_name="s")

    @functools.partial(
        pl.kernel,
        mesh=mesh,
        out_type=jax.ShapeDtypeStruct((B, D), jnp.float32),
        scratch_types=[
            pltpu.VMEM((C,), jnp.int32),
            pltpu.VMEM((C, D), jnp.float32),
            pltpu.SemaphoreType.DMA,
        ],
        compiler_params=pltpu.CompilerParams(use_tc_tiling_on_sc=False),
    )
    def gather_kernel(emb_hbm, idx_hbm, out_hbm, idx_v, rows_v, sem):
        wid = lax.axis_index("s") * NC + lax.axis_index("c")
        base = wid * b_per_w

        def chunk_body(ci, carry):
            off = base + ci * C
            pltpu.sync_copy(idx_hbm.at[pl.ds(off, C)], idx_v)
            pltpu.async_copy(emb_hbm.at[idx_v], rows_v, sem).wait()

            def scale_body(i, c2):
                rows_v[i, pl.ds(0, L)] = rows_v[i, pl.ds(0, L)] * _SCALE
                rows_v[i, pl.ds(L, L)] = rows_v[i, pl.ds(L, L)] * _SCALE
                return c2

            lax.fori_loop(0, C, scale_body, 0, unroll=4)
            pltpu.sync_copy(rows_v, out_hbm.at[pl.ds(off, C)])
            return carry

        lax.fori_loop(0, n_chunks, chunk_body, 0)

    return gather_kernel


def kernel(inputs, emb):
    n, s = inputs.shape
    B = n * s
    idx = inputs.reshape(B)
    out = _make_gather(B)(emb, idx)
    return out.reshape(n, s, D)


# speedup vs baseline: 1.1534x; 1.1534x over previous
import functools
import math

import jax
import jax.numpy as jnp
from jax import lax
from jax.experimental import pallas as pl
from jax.experimental.pallas import tpu as pltpu
from jax.experimental.pallas import tpu_sc as plsc

D = 32
NC, NS, L = 2, 16, 16
NW = NC * NS
C = 128
G = 4
NBUF = 2 * G

_SCALE = math.sqrt(float(D))


@functools.cache
def _make_gather(B):
    assert B % (NW * C) == 0
    b_per_w = B // NW
    n_chunks = b_per_w // C
    assert n_chunks % NBUF == 0 and n_chunks >= NBUF
    n_groups = n_chunks // NBUF
    mesh = plsc.VectorSubcoreMesh(core_axis_name="c", subcore_axis_name="s")

    @functools.partial(
        pl.kernel,
        mesh=mesh,
        out_type=jax.ShapeDtypeStruct((B, D), jnp.float32),
        scratch_types=[
            pltpu.VMEM((n_chunks, C), jnp.int32),
            pltpu.VMEM((NBUF, C, D), jnp.float32),
            pltpu.SemaphoreType.DMA((NBUF,)),
            pltpu.SemaphoreType.DMA((NBUF,)),
        ],
        compiler_params=pltpu.CompilerParams(use_tc_tiling_on_sc=False),
    )
    def gather_kernel(emb_hbm, idx_hbm, out_hbm, idx_v, rows_v, sem_g, sem_o):
        wid = lax.axis_index("s") * NC + lax.axis_index("c")
        base = wid * b_per_w

        pltpu.sync_copy(idx_hbm.at[pl.ds(wid * n_chunks, n_chunks)], idx_v)

        def gather_copy(ci, s):
            return pltpu.make_async_copy(
                emb_hbm.at[idx_v.at[ci]], rows_v.at[s], sem_g.at[s])

        def out_copy(ci, s):
            return pltpu.make_async_copy(
                rows_v.at[s], out_hbm.at[pl.ds(base + ci * C, C)],
                sem_o.at[s])

        for b in range(G):
            gather_copy(b, b).start()

        def group_body(g, carry):
            for b in range(NBUF):
                ci = g * NBUF + b
                gather_copy(ci, b).wait()

                def scale_body(i, c2):
                    rows_v[b, i, pl.ds(0, L)] = rows_v[b, i, pl.ds(0, L)] * _SCALE
                    rows_v[b, i, pl.ds(L, L)] = rows_v[b, i, pl.ds(L, L)] * _SCALE
                    return c2

                lax.fori_loop(0, C, scale_body, 0, unroll=4)
                out_copy(ci, b).start()

                sp = (b + G) % NBUF
                nxt = ci + G

                @pl.when(nxt < n_chunks)
                def _prefetch():
                    @pl.when(nxt >= NBUF)
                    def _drain():
                        out_copy(nxt - NBUF, sp).wait()

                    gather_copy(nxt, sp).start()

            return carry

        lax.fori_loop(0, n_groups, group_body, 0)

        for b in range(NBUF):
            out_copy(n_chunks - NBUF + b, b).wait()

    return gather_kernel


def kernel(inputs, emb):
    n, s = inputs.shape
    B = n * s
    idx = inputs.reshape(B // C, C)
    out = _make_gather(B)(emb, idx)
    return out.reshape(n, s, D)


# speedup vs baseline: 1.4819x; 1.2847x over previous
import functools
import math

import jax
import jax.numpy as jnp
from jax import lax
from jax.experimental import pallas as pl
from jax.experimental.pallas import tpu as pltpu
from jax.experimental.pallas import tpu_sc as plsc

D = 32
NC, NS, L = 2, 16, 16
NW = NC * NS
W = 128
NBUF = 4
G = 2

_SCALE = math.sqrt(float(D))


@functools.cache
def _make_gather(S, B):
    assert B % (NW * W * NBUF) == 0
    b_per_w = B // NW
    n_chunks = S * (b_per_w // W)
    mesh = plsc.VectorSubcoreMesh(core_axis_name="c", subcore_axis_name="s")

    @functools.partial(
        pl.kernel,
        mesh=mesh,
        out_type=jax.ShapeDtypeStruct((S, D, B), jnp.float32),
        scratch_types=[
            pltpu.VMEM((NBUF, W), jnp.int32),
            pltpu.VMEM((NBUF, W, D), jnp.float32),
            pltpu.VMEM((NBUF, D, W), jnp.float32),
            pltpu.SemaphoreType.DMA((NBUF,)),
            pltpu.SemaphoreType.DMA((NBUF,)),
            pltpu.SemaphoreType.DMA((NBUF,)),
        ],
        compiler_params=pltpu.CompilerParams(
            use_tc_tiling_on_sc=False, needs_layout_passes=False),
    )
    def gather_kernel(emb, idx_t, out3, raw_v, g_v, stg_v,
                      sem_i, sem_g, sem_o):
        wid = lax.axis_index("s") * NC + lax.axis_index("c")
        bstripe = wid * b_per_w

        def idx_copy(ci, sl):
            s = ci // NBUF
            b0 = bstripe + (ci % NBUF) * W
            return pltpu.make_async_copy(
                idx_t.at[s, pl.ds(b0, W)], raw_v.at[sl], sem_i.at[sl])

        def gather_copy(sl):
            return pltpu.make_async_copy(
                emb.at[raw_v.at[sl]], g_v.at[sl], sem_g.at[sl])

        def out_copy(ci, sl):
            s = ci // NBUF
            b0 = bstripe + (ci % NBUF) * W
            return pltpu.make_async_copy(
                stg_v.at[sl], out3.at[s, :, pl.ds(b0, W)], sem_o.at[sl])

        for b in range(NBUF):
            idx_copy(b, b).start()
        for b in range(G):
            idx_copy(b, b).wait()
            gather_copy(b).start()

        iota = lax.iota(jnp.int32, L)

        def group_body(g, carry):
            for b in range(NBUF):
                ci = g * NBUF + b
                gather_copy(b).wait()

                @pl.when(ci >= NBUF)
                def _drain():
                    out_copy(ci - NBUF, b).wait()

                def tgrp(k, c2):
                    rows = iota + k * L
                    for d in range(D):
                        cols = jnp.full((L,), d, jnp.int32)
                        vals = plsc.load_gather(g_v.at[b], [rows, cols])
                        stg_v[b, d, pl.ds(k * L, L)] = vals * _SCALE
                    return c2

                lax.fori_loop(0, W // L, tgrp, 0)
                out_copy(ci, b).start()

                nxt = ci + G
                sp = (b + G) % NBUF

                @pl.when(nxt < n_chunks)
                def _prefetch():
                    idx_copy(nxt, sp).wait()
                    gather_copy(sp).start()

                nxt2 = ci + NBUF

                @pl.when(nxt2 < n_chunks)
                def _refill():
                    idx_copy(nxt2, b).start()

            return carry

        lax.fori_loop(0, n_chunks // NBUF, group_body, 0)

        for b in range(NBUF):
            out_copy(n_chunks - NBUF + b, b).wait()

    return gather_kernel


def kernel(inputs, emb):
    n, s = inputs.shape
    raw = _make_gather(s, n)(emb, inputs.T)
    return raw.transpose(2, 0, 1)


# speedup vs baseline: 1.4851x; 1.0022x over previous
import functools
import math

import jax
import jax.numpy as jnp
from jax import lax
from jax.experimental import pallas as pl
from jax.experimental.pallas import tpu as pltpu
from jax.experimental.pallas import tpu_sc as plsc

D = 32
NC, NS, L = 2, 16, 16
NW = NC * NS
W = 128
NBUF = 4
G = 2

_SCALE = math.sqrt(float(D))


@functools.cache
def _make_gather(S, B):
    assert B % (NW * W * NBUF) == 0
    b_per_w = B // NW
    n_chunks = S * (b_per_w // W)
    mesh = plsc.VectorSubcoreMesh(core_axis_name="c", subcore_axis_name="s")

    @functools.partial(
        pl.kernel,
        mesh=mesh,
        out_type=jax.ShapeDtypeStruct((S, D, B), jnp.float32),
        scratch_types=[
            pltpu.VMEM((NBUF, W), jnp.int32),
            pltpu.VMEM((NBUF, W, D), jnp.float32),
            pltpu.VMEM((NBUF, D, W), jnp.float32),
            pltpu.SemaphoreType.DMA((NBUF,)),
            pltpu.SemaphoreType.DMA((NBUF,)),
            pltpu.SemaphoreType.DMA((NBUF,)),
        ],
        compiler_params=pltpu.CompilerParams(
            use_tc_tiling_on_sc=False, needs_layout_passes=False),
    )
    def gather_kernel(emb, idx_t, out3, raw_v, g_v, stg_v,
                      sem_i, sem_g, sem_o):
        wid = lax.axis_index("s") * NC + lax.axis_index("c")
        bstripe = wid * b_per_w

        def idx_copy(ci, sl):
            s = ci // NBUF
            b0 = bstripe + (ci % NBUF) * W
            return pltpu.make_async_copy(
                idx_t.at[s, pl.ds(b0, W)], raw_v.at[sl], sem_i.at[sl])

        def gather_copy(sl):
            return pltpu.make_async_copy(
                emb.at[raw_v.at[sl]], g_v.at[sl], sem_g.at[sl])

        def out_copy(ci, sl):
            s = ci // NBUF
            b0 = bstripe + (ci % NBUF) * W
            return pltpu.make_async_copy(
                stg_v.at[sl], out3.at[s, :, pl.ds(b0, W)], sem_o.at[sl])

        for b in range(NBUF):
            idx_copy(b, b).start()
        for b in range(G):
            idx_copy(b, b).wait()
            gather_copy(b).start()

        iota = lax.iota(jnp.int32, L)
        cols = [jnp.full((L,), d, jnp.int32) for d in range(D)]

        def chunk_body(ci, carry):
            b = ci % NBUF
            gather_copy(b).wait()

            @pl.when(ci >= NBUF)
            def _drain():
                out_copy(ci - NBUF, b).wait()

            for k in range(W // L):
                rows = iota + k * L
                for d in range(D):
                    vals = plsc.load_gather(g_v.at[b], [rows, cols[d]])
                    stg_v[b, d, pl.ds(k * L, L)] = vals * _SCALE

            out_copy(ci, b).start()

            nxt = ci + G
            sp = (b + G) % NBUF

            @pl.when(nxt < n_chunks)
            def _prefetch():
                idx_copy(nxt, sp).wait()
                gather_copy(sp).start()

            nxt2 = ci + NBUF

            @pl.when(nxt2 < n_chunks)
            def _refill():
                idx_copy(nxt2, b).start()

            return carry

        lax.fori_loop(0, n_chunks, chunk_body, 0)

        for b in range(NBUF):
            out_copy(n_chunks - NBUF + b, b).wait()

    return gather_kernel


def kernel(inputs, emb):
    n, s = inputs.shape
    raw = _make_gather(s, n)(emb, inputs.T)
    return raw.transpose(2, 0, 1)


# speedup vs baseline: 2.4294x; 1.6358x over previous
import functools
import math

import jax
import jax.numpy as jnp
from jax import lax
from jax.experimental import pallas as pl
from jax.experimental.pallas import tpu as pltpu
from jax.experimental.pallas import tpu_sc as plsc

D = 32
NC, NS, L = 2, 16, 16
NW = NC * NS
W = 128
NBUF = 4
G = 2

_SCALE = math.sqrt(float(D))


@functools.cache
def _make_gather(S, B):
    assert B % (NW * W * NBUF) == 0
    b_per_w = B // NW
    n_chunks = S * (b_per_w // W)
    mesh = plsc.VectorSubcoreMesh(core_axis_name="c", subcore_axis_name="s")

    @functools.partial(
        pl.kernel,
        mesh=mesh,
        out_type=jax.ShapeDtypeStruct((S, D, B), jnp.float32),
        scratch_types=[
            pltpu.VMEM((NBUF, W), jnp.int32),
            pltpu.VMEM((NBUF, W, D), jnp.float32),
            pltpu.VMEM((NBUF, D, W + 1), jnp.float32),
            pltpu.SemaphoreType.DMA((NBUF,)),
            pltpu.SemaphoreType.DMA((NBUF,)),
            pltpu.SemaphoreType.DMA((NBUF,)),
        ],
        compiler_params=pltpu.CompilerParams(
            use_tc_tiling_on_sc=False, needs_layout_passes=False),
    )
    def gather_kernel(emb, idx_t, out3, raw_v, g_v, stg_v,
                      sem_i, sem_g, sem_o):
        wid = lax.axis_index("s") * NC + lax.axis_index("c")
        bstripe = wid * b_per_w

        def idx_copy(ci, sl):
            s = ci // NBUF
            b0 = bstripe + (ci % NBUF) * W
            return pltpu.make_async_copy(
                idx_t.at[s, pl.ds(b0, W)], raw_v.at[sl], sem_i.at[sl])

        def gather_copy(sl):
            return pltpu.make_async_copy(
                emb.at[raw_v.at[sl]], g_v.at[sl], sem_g.at[sl])

        def out_copy(ci, sl):
            s = ci // NBUF
            b0 = bstripe + (ci % NBUF) * W
            return pltpu.make_async_copy(
                stg_v.at[sl, :, pl.ds(0, W)], out3.at[s, :, pl.ds(b0, W)],
                sem_o.at[sl])

        for b in range(NBUF):
            idx_copy(b, b).start()
        for b in range(G):
            idx_copy(b, b).wait()
            gather_copy(b).start()

        iota = lax.iota(jnp.int32, L)
        r_lo = iota
        r_hi = iota + L

        def chunk_body(ci, carry):
            b = ci % NBUF
            gather_copy(b).wait()

            @pl.when(ci >= NBUF)
            def _drain():
                out_copy(ci - NBUF, b).wait()

            for j in range(W):
                cj = jnp.full((L,), j, jnp.int32)
                v0 = g_v[b, j, pl.ds(0, L)] * _SCALE
                v1 = g_v[b, j, pl.ds(L, L)] * _SCALE
                plsc.store_scatter(stg_v.at[b], [r_lo, cj], v0)
                plsc.store_scatter(stg_v.at[b], [r_hi, cj], v1)

            out_copy(ci, b).start()

            nxt = ci + G
            sp = (b + G) % NBUF

            @pl.when(nxt < n_chunks)
            def _prefetch():
                idx_copy(nxt, sp).wait()
                gather_copy(sp).start()

            nxt2 = ci + NBUF

            @pl.when(nxt2 < n_chunks)
            def _refill():
                idx_copy(nxt2, b).start()

            return carry

        lax.fori_loop(0, n_chunks, chunk_body, 0)

        for b in range(NBUF):
            out_copy(n_chunks - NBUF + b, b).wait()

    return gather_kernel


def kernel(inputs, emb):
    n, s = inputs.shape
    raw = _make_gather(s, n)(emb, inputs.T)
    return raw.transpose(2, 0, 1)


# speedup vs baseline: 2.4367x; 1.0030x over previous
import functools
import math

import jax
import jax.numpy as jnp
from jax import lax
from jax.experimental import pallas as pl
from jax.experimental.pallas import tpu as pltpu
from jax.experimental.pallas import tpu_sc as plsc

D = 32
NC, NS, L = 2, 16, 16
NW = NC * NS
W = 128
NBUF = 4
G = 2

_SCALE = math.sqrt(float(D))


@functools.cache
def _make_detile(S, B):
    assert B % (NW * W * NBUF) == 0
    b_per_w = B // NW
    kpw = b_per_w // W
    nblk = B // W
    mesh = plsc.VectorSubcoreMesh(core_axis_name="c", subcore_axis_name="s")
    ND = 4
    GD = 2

    @functools.partial(
        pl.kernel,
        mesh=mesh,
        out_type=jax.ShapeDtypeStruct((S * nblk, W), jnp.int32),
        scratch_types=[
            pltpu.VMEM((ND, kpw, W), jnp.int32),
            pltpu.SemaphoreType.DMA((ND,)),
            pltpu.SemaphoreType.DMA((ND,)),
        ],
        compiler_params=pltpu.CompilerParams(use_tc_tiling_on_sc=True),
    )
    def detile_kernel(idx_t, out, buf, sem_in, sem_out):
        wid = lax.axis_index("s") * NC + lax.axis_index("c")
        bstripe = wid * b_per_w

        def in_copy(s, sl, k):
            return pltpu.make_async_copy(
                idx_t.at[s, pl.ds(bstripe + k * W, W)], buf.at[sl, k],
                sem_in.at[sl])

        def out_copy(s, sl):
            return pltpu.make_async_copy(
                buf.at[sl], out.at[pl.ds(s * nblk + wid * kpw, kpw)],
                sem_out.at[sl])

        for s in range(GD):
            for k in range(kpw):
                in_copy(s, s % ND, k).start()

        def body(s, carry):
            sl = s % ND
            for k in range(kpw):
                in_copy(s, sl, k).wait()
            out_copy(s, sl).start()
            nxt = s + GD

            @pl.when(nxt < S)
            def _refill():
                @pl.when(nxt >= ND)
                def _drain():
                    out_copy(nxt - ND, nxt % ND).wait()

                for k in range(kpw):
                    in_copy(nxt, nxt % ND, k).start()

            return carry

        lax.fori_loop(0, S, body, 0)

        for s in range(S - ND, S):
            out_copy(s, s % ND).wait()

    return detile_kernel


@functools.cache
def _make_gather(S, B):
    assert B % (NW * W * NBUF) == 0
    b_per_w = B // NW
    n_chunks = S * (b_per_w // W)
    mesh = plsc.VectorSubcoreMesh(core_axis_name="c", subcore_axis_name="s")

    @functools.partial(
        pl.kernel,
        mesh=mesh,
        out_type=jax.ShapeDtypeStruct((S, D, B), jnp.float32),
        scratch_types=[
            pltpu.VMEM((NBUF, W), jnp.int32),
            pltpu.VMEM((NBUF, W, D), jnp.float32),
            pltpu.VMEM((NBUF, D, W + 1), jnp.float32),
            pltpu.SemaphoreType.DMA((NBUF,)),
            pltpu.SemaphoreType.DMA((NBUF,)),
            pltpu.SemaphoreType.DMA((NBUF,)),
        ],
        compiler_params=pltpu.CompilerParams(
            use_tc_tiling_on_sc=False, needs_layout_passes=False),
    )
    def gather_kernel(emb, idx_lin, out3, raw_v, g_v, stg_v,
                      sem_i, sem_g, sem_o):
        wid = lax.axis_index("s") * NC + lax.axis_index("c")
        bstripe = wid * b_per_w
        nblk = B // W
        kpw = b_per_w // W

        def idx_copy(ci, sl):
            row = (ci // NBUF) * nblk + wid * kpw + (ci % NBUF)
            return pltpu.make_async_copy(
                idx_lin.at[row], raw_v.at[sl], sem_i.at[sl])

        def gather_copy(sl):
            return pltpu.make_async_copy(
                emb.at[raw_v.at[sl]], g_v.at[sl], sem_g.at[sl])

        def out_copy(ci, sl):
            s = ci // NBUF
            b0 = bstripe + (ci % NBUF) * W
            return pltpu.make_async_copy(
                stg_v.at[sl, :, pl.ds(0, W)], out3.at[s, :, pl.ds(b0, W)],
                sem_o.at[sl])

        for b in range(NBUF):
            idx_copy(b, b).start()
        for b in range(G):
            idx_copy(b, b).wait()
            gather_copy(b).start()

        iota = lax.iota(jnp.int32, L)
        r_lo = iota
        r_hi = iota + L

        def chunk_body(ci, carry):
            b = ci % NBUF
            gather_copy(b).wait()

            @pl.when(ci >= NBUF)
            def _drain():
                out_copy(ci - NBUF, b).wait()

            for j in range(W):
                cj = jnp.full((L,), j, jnp.int32)
                v0 = g_v[b, j, pl.ds(0, L)] * _SCALE
                v1 = g_v[b, j, pl.ds(L, L)] * _SCALE
                plsc.store_scatter(stg_v.at[b], [r_lo, cj], v0)
                plsc.store_scatter(stg_v.at[b], [r_hi, cj], v1)

            out_copy(ci, b).start()

            nxt = ci + G
            sp = (b + G) % NBUF

            @pl.when(nxt < n_chunks)
            def _prefetch():
                idx_copy(nxt, sp).wait()
                gather_copy(sp).start()

            nxt2 = ci + NBUF

            @pl.when(nxt2 < n_chunks)
            def _refill():
                idx_copy(nxt2, b).start()

            return carry

        lax.fori_loop(0, n_chunks, chunk_body, 0)

        for b in range(NBUF):
            out_copy(n_chunks - NBUF + b, b).wait()

    return gather_kernel


def kernel(inputs, emb):
    n, s = inputs.shape
    idx_lin = _make_detile(s, n)(inputs.T)
    raw = _make_gather(s, n)(emb, idx_lin)
    return raw.transpose(2, 0, 1)
